# baseline (speedup 1.0000x reference)
import jax
import jax.numpy as jnp
from jax import lax
from jax.experimental import pallas as pl
from jax.experimental.pallas import tpu as pltpu
from jax.experimental.pallas import tpu_sc as plsc

N_POINTS = 1048576
PLANE = 512
FDIM = 64
OUT_DIM = 3 * FDIM
ROWS_PER_PLANE = PLANE * PLANE

NC = 2
NS = 16
NW = NC * NS
PPW = N_POINTS // NW
C = 128
LANES = 16


def _sc_body(x_hbm, tab_hbm, out_hbm, xv, xidx, idxv, wv, rows, outv, sem):
    cid = lax.axis_index("c")
    sid = lax.axis_index("s")
    wid = sid * NC + cid
    base = wid * PPW

    def chunk_body(g, carry):
        start = base + g * C

        def xidx_body(v, carry2):
            sl = pl.ds(v * LANES, LANES)
            tri = lax.iota(jnp.int32, LANES) * 3 + (start + v * LANES) * 3
            xidx[0, sl] = tri
            xidx[1, sl] = tri + 1
            xidx[2, sl] = tri + 2
            return carry2

        lax.fori_loop(0, C // LANES, xidx_body, 0)
        xdescs = [
            pltpu.async_copy(x_hbm.at[xidx.at[d]], xv.at[d], sem)
            for d in range(3)
        ]
        for d in xdescs:
            d.wait()

        def vec_body(v, carry2):
            off = v * LANES
            sl = pl.ds(off, LANES)
            c0 = xv[0, sl]
            c1 = xv[1, sl]
            c2 = xv[2, sl]
            for i, (cw, ch) in enumerate(((c1, c2), (c0, c2), (c0, c1))):
                gx = cw * 2.0 - 1.0
                gy = ch * 2.0 - 1.0
                ix = (gx + 1.0) * 0.5 * float(PLANE - 1)
                iy = (gy + 1.0) * 0.5 * float(PLANE - 1)
                ix = jnp.clip(ix, 0.0, float(PLANE - 1))
                iy = jnp.clip(iy, 0.0, float(PLANE - 1))
                x0 = ix.astype(jnp.int32)
                y0 = iy.astype(jnp.int32)
                wx = ix - x0.astype(jnp.float32)
                wy = iy - y0.astype(jnp.float32)
                x1 = jnp.minimum(x0 + 1, PLANE - 1)
                y1 = jnp.minimum(y0 + 1, PLANE - 1)
                r0 = y0 * PLANE + (i * ROWS_PER_PLANE)
                r1 = y1 * PLANE + (i * ROWS_PER_PLANE)
                idxv[4 * i + 0, sl] = r0 + x0
                idxv[4 * i + 1, sl] = r0 + x1
                idxv[4 * i + 2, sl] = r1 + x0
                idxv[4 * i + 3, sl] = r1 + x1
                ox = 1.0 - wx
                oy = 1.0 - wy
                wbase = ((i * (C // LANES) + v) * 4) * LANES
                wv[pl.ds(wbase + 0 * LANES, LANES)] = ox * oy
                wv[pl.ds(wbase + 1 * LANES, LANES)] = wx * oy
                wv[pl.ds(wbase + 2 * LANES, LANES)] = ox * wy
                wv[pl.ds(wbase + 3 * LANES, LANES)] = wx * wy
            return carry2

        lax.fori_loop(0, C // LANES, vec_body, 0)

        for i in range(3):
            descs = [
                pltpu.async_copy(tab_hbm.at[idxv.at[4 * i + cc]], rows.at[cc], sem)
                for cc in range(4)
            ]
            for d in descs:
                d.wait()

            def blend_group(v, carry2, i=i):
                wbase = ((i * (C // LANES) + v) * 4) * LANES
                w00v = wv[pl.ds(wbase + 0 * LANES, LANES)]
                w01v = wv[pl.ds(wbase + 1 * LANES, LANES)]
                w10v = wv[pl.ds(wbase + 2 * LANES, LANES)]
                w11v = wv[pl.ds(wbase + 3 * LANES, LANES)]
                for j in range(LANES):
                    p = v * LANES + j
                    w00 = w00v[j]
                    w01 = w01v[j]
                    w10 = w10v[j]
                    w11 = w11v[j]
                    for f in range(FDIM // LANES):
                        sl = pl.ds(f * LANES, LANES)
                        acc = (rows[0, p, sl] * w00 + rows[1, p, sl] * w01
                               + rows[2, p, sl] * w10 + rows[3, p, sl] * w11)
                        outv[p, pl.ds(i * FDIM + f * LANES, LANES)] = acc
                return carry2

            lax.fori_loop(0, C // LANES, blend_group, 0)

        pltpu.sync_copy(outv, out_hbm.at[pl.ds(start, C)])
        return carry

    lax.fori_loop(0, PPW // C, chunk_body, 0)


def _run(x, fm):
    xflat = x.reshape(N_POINTS * 3)
    tab = fm.reshape(3 * ROWS_PER_PLANE, FDIM)
    mesh = plsc.VectorSubcoreMesh(core_axis_name="c", subcore_axis_name="s")
    kfn = pl.kernel(
        _sc_body,
        out_type=jax.ShapeDtypeStruct((N_POINTS, OUT_DIM), jnp.float32),
        mesh=mesh,
        scratch_types=[
            pltpu.VMEM((3, C), jnp.float32),
            pltpu.VMEM((3, C), jnp.int32),
            pltpu.VMEM((12, C), jnp.int32),
            pltpu.VMEM((3 * (C // LANES) * 4 * LANES,), jnp.float32),
            pltpu.VMEM((4, C, FDIM), jnp.float32),
            pltpu.VMEM((C, OUT_DIM), jnp.float32),
            pltpu.SemaphoreType.DMA,
        ],
        compiler_params=pltpu.CompilerParams(use_tc_tiling_on_sc=False),
    )
    return kfn(xflat, tab)


def kernel(x, level, fm):
    del level
    return _run(x, fm)


# speedup vs baseline: 1.3096x; 1.3096x over previous
import jax
import jax.numpy as jnp
from jax import lax
from jax.experimental import pallas as pl
from jax.experimental.pallas import tpu as pltpu
from jax.experimental.pallas import tpu_sc as plsc

N_POINTS = 1048576
PLANE = 512
FDIM = 64
OUT_DIM = 3 * FDIM
ROWS_PER_PLANE = PLANE * PLANE

NC = 2
NS = 16
NW = NC * NS
PPW = N_POINTS // NW
C = 128
LANES = 16
NCHUNK = PPW // C


def _sc_body(x_hbm, tab_hbm, out_hbm, xv, idxv, wv, rows, outv, sems, semo):
    cid = lax.axis_index("c")
    sid = lax.axis_index("s")
    wid = sid * NC + cid
    base = wid * PPW

    def chunk_body(g, carry):
        start = base + g * C
        pltpu.sync_copy(x_hbm.at[:, pl.ds(start, C)], xv)

        def vec_body(v, carry2):
            off = v * LANES
            sl = pl.ds(off, LANES)
            c0 = xv[0, sl]
            c1 = xv[1, sl]
            c2 = xv[2, sl]
            for i, (cw, ch) in enumerate(((c1, c2), (c0, c2), (c0, c1))):
                gx = cw * 2.0 - 1.0
                gy = ch * 2.0 - 1.0
                ix = (gx + 1.0) * 0.5 * float(PLANE - 1)
                iy = (gy + 1.0) * 0.5 * float(PLANE - 1)
                ix = jnp.clip(ix, 0.0, float(PLANE - 1))
                iy = jnp.clip(iy, 0.0, float(PLANE - 1))
                x0 = ix.astype(jnp.int32)
                y0 = iy.astype(jnp.int32)
                wx = ix - x0.astype(jnp.float32)
                wy = iy - y0.astype(jnp.float32)
                x1 = jnp.minimum(x0 + 1, PLANE - 1)
                y1 = jnp.minimum(y0 + 1, PLANE - 1)
                r0 = y0 * PLANE + (i * ROWS_PER_PLANE)
                r1 = y1 * PLANE + (i * ROWS_PER_PLANE)
                idxv[4 * i + 0, sl] = r0 + x0
                idxv[4 * i + 1, sl] = r0 + x1
                idxv[4 * i + 2, sl] = r1 + x0
                idxv[4 * i + 3, sl] = r1 + x1
                ox = 1.0 - wx
                oy = 1.0 - wy
                wbase = ((i * (C // LANES) + v) * 4) * LANES
                wv[pl.ds(wbase + 0 * LANES, LANES)] = ox * oy
                wv[pl.ds(wbase + 1 * LANES, LANES)] = wx * oy
                wv[pl.ds(wbase + 2 * LANES, LANES)] = ox * wy
                wv[pl.ds(wbase + 3 * LANES, LANES)] = wx * wy
            return carry2

        lax.fori_loop(0, C // LANES, vec_body, 0)

        descs = []
        for i in range(3):
            descs.append([
                pltpu.async_copy(
                    tab_hbm.at[idxv.at[4 * i + cc]], rows.at[4 * i + cc],
                    sems.at[i])
                for cc in range(4)
            ])

        @pl.when(g > 0)
        def _():
            pltpu.make_async_copy(
                outv, out_hbm.at[pl.ds(base + (g - 1) * C, C)], semo).wait()

        for i in range(3):
            for d in descs[i]:
                d.wait()

            def blend_group(v, carry2, i=i):
                wbase = ((i * (C // LANES) + v) * 4) * LANES
                w00v = wv[pl.ds(wbase + 0 * LANES, LANES)]
                w01v = wv[pl.ds(wbase + 1 * LANES, LANES)]
                w10v = wv[pl.ds(wbase + 2 * LANES, LANES)]
                w11v = wv[pl.ds(wbase + 3 * LANES, LANES)]
                for j in range(LANES):
                    p = v * LANES + j
                    w00 = w00v[j]
                    w01 = w01v[j]
                    w10 = w10v[j]
                    w11 = w11v[j]
                    for f in range(FDIM // LANES):
                        sl = pl.ds(f * LANES, LANES)
                        acc = (rows[4 * i + 0, p, sl] * w00
                               + rows[4 * i + 1, p, sl] * w01
                               + rows[4 * i + 2, p, sl] * w10
                               + rows[4 * i + 3, p, sl] * w11)
                        outv[p, pl.ds(i * FDIM + f * LANES, LANES)] = acc
                return carry2

            lax.fori_loop(0, C // LANES, blend_group, 0)

        pltpu.async_copy(outv, out_hbm.at[pl.ds(start, C)], semo)
        return carry

    lax.fori_loop(0, NCHUNK, chunk_body, 0)
    pltpu.make_async_copy(
        outv, out_hbm.at[pl.ds(base + (NCHUNK - 1) * C, C)], semo).wait()


def _run(x, fm):
    xT = x.T
    tab = fm.reshape(3 * ROWS_PER_PLANE, FDIM)
    mesh = plsc.VectorSubcoreMesh(core_axis_name="c", subcore_axis_name="s")
    kfn = pl.kernel(
        _sc_body,
        out_type=jax.ShapeDtypeStruct((N_POINTS, OUT_DIM), jnp.float32),
        mesh=mesh,
        scratch_types=[
            pltpu.VMEM((3, C), jnp.float32),
            pltpu.VMEM((12, C), jnp.int32),
            pltpu.VMEM((3 * (C // LANES) * 4 * LANES,), jnp.float32),
            pltpu.VMEM((12, C, FDIM), jnp.float32),
            pltpu.VMEM((C, OUT_DIM), jnp.float32),
            pltpu.SemaphoreType.DMA((3,)),
            pltpu.SemaphoreType.DMA,
        ],
        compiler_params=pltpu.CompilerParams(use_tc_tiling_on_sc=False),
    )
    return kfn(xT, tab)


def kernel(x, level, fm):
    del level
    return _run(x, fm)


# speedup vs baseline: 1.3879x; 1.0598x over previous
import jax
import jax.numpy as jnp
from jax import lax
from jax.experimental import pallas as pl
from jax.experimental.pallas import tpu as pltpu
from jax.experimental.pallas import tpu_sc as plsc

N_POINTS = 1048576
PLANE = 512
FDIM = 64
OUT_DIM = 3 * FDIM
ROWS_PER_PLANE = PLANE * PLANE

NC = 2
NS = 16
NW = NC * NS
PPW = N_POINTS // NW
C = 128
LANES = 16
NCHUNK = PPW // C


def _sc_body(x_hbm, tab_hbm, out_hbm, xv, idxv, wv, rows, outv, sems, semo):
    cid = lax.axis_index("c")
    sid = lax.axis_index("s")
    wid = sid * NC + cid
    base = wid * PPW

    def chunk_body(g, carry):
        start = base + g * C
        pltpu.sync_copy(x_hbm.at[:, pl.ds(start, C)], xv)

        @plsc.parallel_loop(0, C // LANES)
        def vec_body(v):
            off = v * LANES
            sl = pl.ds(off, LANES)
            lo, hi, w1, ow, sh0, sh1 = [], [], [], [], [], []
            for d in range(3):
                c = xv[d, sl]
                gc = c * 2.0 - 1.0
                t = (gc + 1.0) * 0.5 * float(PLANE - 1)
                t = jnp.clip(t, 0.0, float(PLANE - 1))
                i0 = t.astype(jnp.int32)
                w = t - i0.astype(jnp.float32)
                lo.append(i0)
                hi.append(jnp.minimum(i0 + 1, PLANE - 1))
                w1.append(w)
                ow.append(1.0 - w)
                sh0.append(i0 * PLANE)
                sh1.append(jnp.minimum(i0 + 1, PLANE - 1) * PLANE)
            for i, (dw, dh) in enumerate(((1, 2), (0, 2), (0, 1))):
                r0 = sh0[dh] + (i * ROWS_PER_PLANE)
                r1 = sh1[dh] + (i * ROWS_PER_PLANE)
                idxv[4 * i + 0, sl] = r0 + lo[dw]
                idxv[4 * i + 1, sl] = r0 + hi[dw]
                idxv[4 * i + 2, sl] = r1 + lo[dw]
                idxv[4 * i + 3, sl] = r1 + hi[dw]
                wbase = ((i * (C // LANES) + v) * 4) * LANES
                wv[pl.ds(wbase + 0 * LANES, LANES)] = ow[dw] * ow[dh]
                wv[pl.ds(wbase + 1 * LANES, LANES)] = w1[dw] * ow[dh]
                wv[pl.ds(wbase + 2 * LANES, LANES)] = ow[dw] * w1[dh]
                wv[pl.ds(wbase + 3 * LANES, LANES)] = w1[dw] * w1[dh]

        descs = []
        for i in range(3):
            descs.append([
                pltpu.async_copy(
                    tab_hbm.at[idxv.at[4 * i + cc]], rows.at[4 * i + cc],
                    sems.at[i])
                for cc in range(4)
            ])

        @pl.when(g > 0)
        def _():
            pltpu.make_async_copy(
                outv, out_hbm.at[pl.ds(base + (g - 1) * C, C)], semo).wait()

        for i in range(3):
            for d in descs[i]:
                d.wait()

            @plsc.parallel_loop(0, C // LANES)
            def blend_group(v, i=i):
                wbase = ((i * (C // LANES) + v) * 4) * LANES
                w00v = wv[pl.ds(wbase + 0 * LANES, LANES)]
                w01v = wv[pl.ds(wbase + 1 * LANES, LANES)]
                w10v = wv[pl.ds(wbase + 2 * LANES, LANES)]
                w11v = wv[pl.ds(wbase + 3 * LANES, LANES)]
                for j in range(LANES):
                    p = v * LANES + j
                    w00 = w00v[j]
                    w01 = w01v[j]
                    w10 = w10v[j]
                    w11 = w11v[j]
                    for f in range(FDIM // LANES):
                        sl = pl.ds(f * LANES, LANES)
                        acc = (rows[4 * i + 0, p, sl] * w00
                               + rows[4 * i + 1, p, sl] * w01
                               + rows[4 * i + 2, p, sl] * w10
                               + rows[4 * i + 3, p, sl] * w11)
                        outv[p, pl.ds(i * FDIM + f * LANES, LANES)] = acc

        pltpu.async_copy(outv, out_hbm.at[pl.ds(start, C)], semo)
        return carry

    lax.fori_loop(0, NCHUNK, chunk_body, 0)
    pltpu.make_async_copy(
        outv, out_hbm.at[pl.ds(base + (NCHUNK - 1) * C, C)], semo).wait()


def _run(x, fm):
    xT = x.T
    tab = fm.reshape(3 * ROWS_PER_PLANE, FDIM)
    mesh = plsc.VectorSubcoreMesh(core_axis_name="c", subcore_axis_name="s")
    kfn = pl.kernel(
        _sc_body,
        out_type=jax.ShapeDtypeStruct((N_POINTS, OUT_DIM), jnp.float32),
        mesh=mesh,
        scratch_types=[
            pltpu.VMEM((3, C), jnp.float32),
            pltpu.VMEM((12, C), jnp.int32),
            pltpu.VMEM((3 * (C // LANES) * 4 * LANES,), jnp.float32),
            pltpu.VMEM((12, C, FDIM), jnp.float32),
            pltpu.VMEM((C, OUT_DIM), jnp.float32),
            pltpu.SemaphoreType.DMA((3,)),
            pltpu.SemaphoreType.DMA,
        ],
        compiler_params=pltpu.CompilerParams(use_tc_tiling_on_sc=False),
    )
    return kfn(xT, tab)


def kernel(x, level, fm):
    del level
    return _run(x, fm)
